# Initial kernel scaffold; baseline (speedup 1.0000x reference)
import jax
import jax.numpy as jnp
from jax.experimental import pallas as pl


def kernel(x, mask_prev, enc_W, enc_b, dec_src_W, dec_src_b, dec_self_W, dec_self_b, decoder_type):
    raise NotImplementedError("write your pallas kernel here")


# speedup vs baseline: 13.5371x; 13.5371x over previous
import jax
import jax.numpy as jnp
from jax.experimental import pallas as pl


def _body(x_ref, mp_ref, ew_ref, eb_ref, dw_ref, db_ref, out_ref, nm_ref):
    h = jnp.dot(x_ref[...], ew_ref[...],
                preferred_element_type=jnp.float32) + eb_ref[...]
    mp = mp_ref[...]
    h = jnp.where(mp != 0.0, 0.0, h)
    e = h * h
    ebits = jax.lax.bitcast_convert_type(e, jnp.int32)

    rows = ebits.shape[0]
    t0 = jnp.zeros((rows, 1), jnp.int32)

    def step(i, carry):
        t64, t128 = carry
        bit = jnp.int32(1) << (30 - i)
        c64 = t64 | bit
        c128 = t128 | bit
        cnt64 = jnp.sum((ebits >= c64).astype(jnp.float32), axis=1,
                        keepdims=True)
        cnt128 = jnp.sum((ebits >= c128).astype(jnp.float32), axis=1,
                         keepdims=True)
        t64 = jnp.where(cnt64 >= 64.0, c64, t64)
        t128 = jnp.where(cnt128 >= 128.0, c128, t128)
        return t64, t128

    t64, t128 = jax.lax.fori_loop(0, 31, step, (t0, t0))

    mask_cur = (ebits >= t64).astype(jnp.float32)
    nm_ref[...] = mp + mask_cur
    h_sel = jnp.where(ebits >= t128, h, 0.0)
    out_ref[...] = jnp.dot(h_sel, dw_ref[...],
                           preferred_element_type=jnp.float32) + db_ref[...]


def kernel(x, mask_prev, enc_W, enc_b, dec_src_W, dec_src_b,
           dec_self_W, dec_self_b, decoder_type):
    B, T, IDIM = x.shape
    HDIM = enc_W.shape[0]
    ODIM = dec_src_W.shape[0]
    BT = B * T

    is_src = jnp.asarray(decoder_type) == 1
    dec_W = jnp.where(is_src, dec_src_W, dec_self_W)
    dec_b = jnp.where(is_src, dec_src_b, dec_self_b)

    x2 = x.reshape(BT, IDIM)
    mp2 = mask_prev.reshape(BT, HDIM)
    enc_WT = enc_W.T
    dec_WT = dec_W.T

    R = 256
    grid = (BT // R,)

    out2, nm2 = pl.pallas_call(
        _body,
        grid=grid,
        in_specs=[
            pl.BlockSpec((R, IDIM), lambda i: (i, 0)),
            pl.BlockSpec((R, HDIM), lambda i: (i, 0)),
            pl.BlockSpec((IDIM, HDIM), lambda i: (0, 0)),
            pl.BlockSpec((1, HDIM), lambda i: (0, 0)),
            pl.BlockSpec((HDIM, ODIM), lambda i: (0, 0)),
            pl.BlockSpec((1, ODIM), lambda i: (0, 0)),
        ],
        out_specs=[
            pl.BlockSpec((R, ODIM), lambda i: (i, 0)),
            pl.BlockSpec((R, HDIM), lambda i: (i, 0)),
        ],
        out_shape=[
            jax.ShapeDtypeStruct((BT, ODIM), jnp.float32),
            jax.ShapeDtypeStruct((BT, HDIM), jnp.float32),
        ],
    )(x2, mp2, enc_WT, enc_b.reshape(1, HDIM), dec_WT, dec_b.reshape(1, ODIM))

    return out2.reshape(B, T, ODIM), nm2.reshape(B, T, HDIM)
